# Initial kernel scaffold; baseline (speedup 1.0000x reference)
import jax
import jax.numpy as jnp
from jax import lax
from jax.experimental import pallas as pl

B = 8
N = 1000
C = 81
K = 100
MIN_CONF = 0.7
NMS_THR = 0.3


def _dense_body(rois_ref, probs_ref, flat_ref, out_ref):
    probs = probs_ref[0]
    rois = rois_ref[0]
    flat = flat_ref[0]

    m = jnp.max(probs, axis=1, keepdims=True)
    iota_c = lax.broadcasted_iota(jnp.int32, (N, C), 1)
    cid = jnp.min(jnp.where(probs == m, iota_c, C), axis=1, keepdims=True)

    iota_f = lax.broadcasted_iota(jnp.int32, (N, 4 * C), 1)
    base = cid * 4
    d = []
    for k in range(4):
        mk = iota_f == (base + k)
        d.append(jnp.sum(jnp.where(mk, flat, 0.0), axis=1, keepdims=True))
    dy = d[0] * 0.1
    dx = d[1] * 0.1
    dh = d[2] * 0.2
    dw = d[3] * 0.2

    ry1 = rois[:, 0:1]
    rx1 = rois[:, 1:2]
    ry2 = rois[:, 2:3]
    rx2 = rois[:, 3:4]
    h = ry2 - ry1
    w = rx2 - rx1
    cy = ry1 + 0.5 * h
    cx = rx1 + 0.5 * w
    cy = cy + dy * h
    cx = cx + dx * w
    h = h * jnp.exp(dh)
    w = w * jnp.exp(dw)
    y1 = jnp.clip(cy - 0.5 * h, 0.0, 1.0)
    x1 = jnp.clip(cx - 0.5 * w, 0.0, 1.0)
    y2 = jnp.clip(cy + 0.5 * h, 0.0, 1.0)
    x2 = jnp.clip(cx + 0.5 * w, 0.0, 1.0)

    valid = (cid > 0) & (m >= MIN_CONF)
    sc = jnp.where(valid, m, -1.0)
    clsf = cid.astype(jnp.float32)

    rows = jnp.concatenate(
        [sc.T, y1.T, x1.T, y2.T, x2.T, clsf.T], axis=0)
    out_ref[0] = rows


def _nms_body(data_ref, osc_ref, oy1_ref, ox1_ref, oy2_ref, ox2_ref,
              ocl_ref):
    sc = data_ref[:, 0, :]
    y1 = data_ref[:, 1, :]
    x1 = data_ref[:, 2, :]
    y2 = data_ref[:, 3, :]
    x2 = data_ref[:, 4, :]
    cls = data_ref[:, 5, :]

    off = cls * 10.0
    ny1 = y1 + off
    nx1 = x1 + off
    ny2 = y2 + off
    nx2 = x2 + off
    area = (ny2 - ny1) * (nx2 - nx1)

    iota_n = lax.broadcasted_iota(jnp.int32, (B, N), 1)
    iota_k = lax.broadcasted_iota(jnp.int32, (B, K), 1)
    zeros_k = jnp.zeros((B, K), jnp.float32)

    def ext(sel, a):
        return jnp.sum(jnp.where(sel, a, 0.0), axis=1, keepdims=True)

    def body(k, carry):
        alive, osc, oy1, ox1, oy2, ox2, ocl = carry
        m = jnp.max(alive, axis=1, keepdims=True)
        act = m > 0.0
        isel = jnp.min(jnp.where(alive == m, iota_n, N), axis=1,
                       keepdims=True)
        sel = iota_n == isel
        vy1 = ext(sel, y1)
        vx1 = ext(sel, x1)
        vy2 = ext(sel, y2)
        vx2 = ext(sel, x2)
        vcl = ext(sel, cls)
        sy1 = ext(sel, ny1)
        sx1 = ext(sel, nx1)
        sy2 = ext(sel, ny2)
        sx2 = ext(sel, nx2)
        sarea = (sy2 - sy1) * (sx2 - sx1)
        yy1 = jnp.maximum(ny1, sy1)
        xx1 = jnp.maximum(nx1, sx1)
        yy2 = jnp.minimum(ny2, sy2)
        xx2 = jnp.minimum(nx2, sx2)
        inter = jnp.maximum(yy2 - yy1, 0.0) * jnp.maximum(xx2 - xx1, 0.0)
        union = area + sarea - inter
        iou = inter / jnp.maximum(union, 1e-8)
        supp = (iou > NMS_THR) | sel
        alive = jnp.where(act & supp, -1.0, alive)
        upd = act & (iota_k == k)
        osc = jnp.where(upd, m, osc)
        oy1 = jnp.where(upd, vy1, oy1)
        ox1 = jnp.where(upd, vx1, ox1)
        oy2 = jnp.where(upd, vy2, oy2)
        ox2 = jnp.where(upd, vx2, ox2)
        ocl = jnp.where(upd, vcl, ocl)
        return (alive, osc, oy1, ox1, oy2, ox2, ocl)

    carry = (sc, zeros_k, zeros_k, zeros_k, zeros_k, zeros_k, zeros_k)
    carry = lax.fori_loop(0, K, body, carry)
    _, osc, oy1, ox1, oy2, ox2, ocl = carry
    osc_ref[...] = osc
    oy1_ref[...] = oy1
    ox1_ref[...] = ox1
    oy2_ref[...] = oy2
    ox2_ref[...] = ox2
    ocl_ref[...] = ocl


def kernel(rois, mrcnn_class, mrcnn_bbox):
    flat = mrcnn_bbox.reshape(B, N, 4 * C)
    data = pl.pallas_call(
        _dense_body,
        grid=(B,),
        in_specs=[
            pl.BlockSpec((1, N, 4), lambda b: (b, 0, 0)),
            pl.BlockSpec((1, N, C), lambda b: (b, 0, 0)),
            pl.BlockSpec((1, N, 4 * C), lambda b: (b, 0, 0)),
        ],
        out_specs=pl.BlockSpec((1, 6, N), lambda b: (b, 0, 0)),
        out_shape=jax.ShapeDtypeStruct((B, 6, N), jnp.float32),
    )(rois, mrcnn_class, flat)

    outs = pl.pallas_call(
        _nms_body,
        out_shape=[jax.ShapeDtypeStruct((B, K), jnp.float32)] * 6,
    )(data)
    osc, oy1, ox1, oy2, ox2, ocl = outs
    return jnp.stack([oy1, ox1, oy2, ox2, ocl, osc], axis=-1)


# speedup vs baseline: 17.1181x; 17.1181x over previous
import jax
import jax.numpy as jnp
from jax import lax
from jax.experimental import pallas as pl

B = 8
N = 1000
C = 81
K = 100
MIN_CONF = 0.7
NMS_THR = 0.3


def _dense_body(rois_ref, probs_ref, flat_ref, out_ref):
    probs = probs_ref[0]
    rois = rois_ref[0]
    flat = flat_ref[0]

    m = jnp.max(probs, axis=1, keepdims=True)
    iota_c = lax.broadcasted_iota(jnp.int32, (N, C), 1)
    cid = jnp.min(jnp.where(probs == m, iota_c, C), axis=1, keepdims=True)

    iota_f = lax.broadcasted_iota(jnp.int32, (N, 4 * C), 1)
    base = cid * 4
    d = []
    for k in range(4):
        mk = iota_f == (base + k)
        d.append(jnp.sum(jnp.where(mk, flat, 0.0), axis=1, keepdims=True))
    dy = d[0] * 0.1
    dx = d[1] * 0.1
    dh = d[2] * 0.2
    dw = d[3] * 0.2

    ry1 = rois[:, 0:1]
    rx1 = rois[:, 1:2]
    ry2 = rois[:, 2:3]
    rx2 = rois[:, 3:4]
    h = ry2 - ry1
    w = rx2 - rx1
    cy = ry1 + 0.5 * h
    cx = rx1 + 0.5 * w
    cy = cy + dy * h
    cx = cx + dx * w
    h = h * jnp.exp(dh)
    w = w * jnp.exp(dw)
    y1 = jnp.clip(cy - 0.5 * h, 0.0, 1.0)
    x1 = jnp.clip(cx - 0.5 * w, 0.0, 1.0)
    y2 = jnp.clip(cy + 0.5 * h, 0.0, 1.0)
    x2 = jnp.clip(cx + 0.5 * w, 0.0, 1.0)

    valid = (cid > 0) & (m >= MIN_CONF)
    sc = jnp.where(valid, m, -1.0)
    clsf = cid.astype(jnp.float32)

    cols = jnp.concatenate(
        [sc, y1, x1, y2, x2, clsf], axis=1)
    out_ref[0] = cols


def _nms_body(data_ref, osc_ref, oy1_ref, ox1_ref, oy2_ref, ox2_ref,
              ocl_ref):
    sc = data_ref[:, 0, :]
    y1 = data_ref[:, 1, :]
    x1 = data_ref[:, 2, :]
    y2 = data_ref[:, 3, :]
    x2 = data_ref[:, 4, :]
    cls = data_ref[:, 5, :]

    off = cls * 10.0
    ny1 = y1 + off
    nx1 = x1 + off
    ny2 = y2 + off
    nx2 = x2 + off
    area = (ny2 - ny1) * (nx2 - nx1)

    iota_n = lax.broadcasted_iota(jnp.int32, (B, N), 1)
    iota_k = lax.broadcasted_iota(jnp.int32, (B, K), 1)
    zeros_k = jnp.zeros((B, K), jnp.float32)

    def ext(sel, a):
        return jnp.sum(jnp.where(sel, a, 0.0), axis=1, keepdims=True)

    def body(k, carry):
        alive, osc, oy1, ox1, oy2, ox2, ocl = carry
        m = jnp.max(alive, axis=1, keepdims=True)
        act = m > 0.0
        isel = jnp.min(jnp.where(alive == m, iota_n, N), axis=1,
                       keepdims=True)
        sel = iota_n == isel
        vy1 = ext(sel, y1)
        vx1 = ext(sel, x1)
        vy2 = ext(sel, y2)
        vx2 = ext(sel, x2)
        vcl = ext(sel, cls)
        sy1 = ext(sel, ny1)
        sx1 = ext(sel, nx1)
        sy2 = ext(sel, ny2)
        sx2 = ext(sel, nx2)
        sarea = (sy2 - sy1) * (sx2 - sx1)
        yy1 = jnp.maximum(ny1, sy1)
        xx1 = jnp.maximum(nx1, sx1)
        yy2 = jnp.minimum(ny2, sy2)
        xx2 = jnp.minimum(nx2, sx2)
        inter = jnp.maximum(yy2 - yy1, 0.0) * jnp.maximum(xx2 - xx1, 0.0)
        union = area + sarea - inter
        iou = inter / jnp.maximum(union, 1e-8)
        supp = (iou > NMS_THR) | sel
        alive = jnp.where(act & supp, -1.0, alive)
        upd = act & (iota_k == k)
        osc = jnp.where(upd, m, osc)
        oy1 = jnp.where(upd, vy1, oy1)
        ox1 = jnp.where(upd, vx1, ox1)
        oy2 = jnp.where(upd, vy2, oy2)
        ox2 = jnp.where(upd, vx2, ox2)
        ocl = jnp.where(upd, vcl, ocl)
        return (alive, osc, oy1, ox1, oy2, ox2, ocl)

    carry = (sc, zeros_k, zeros_k, zeros_k, zeros_k, zeros_k, zeros_k)
    carry = lax.fori_loop(0, K, body, carry)
    _, osc, oy1, ox1, oy2, ox2, ocl = carry
    osc_ref[...] = osc
    oy1_ref[...] = oy1
    ox1_ref[...] = ox1
    oy2_ref[...] = oy2
    ox2_ref[...] = ox2
    ocl_ref[...] = ocl


def kernel(rois, mrcnn_class, mrcnn_bbox):
    flat = mrcnn_bbox.reshape(B, N, 4 * C)
    data = pl.pallas_call(
        _dense_body,
        grid=(B,),
        in_specs=[
            pl.BlockSpec((1, N, 4), lambda b: (b, 0, 0)),
            pl.BlockSpec((1, N, C), lambda b: (b, 0, 0)),
            pl.BlockSpec((1, N, 4 * C), lambda b: (b, 0, 0)),
        ],
        out_specs=pl.BlockSpec((1, N, 6), lambda b: (b, 0, 0)),
        out_shape=jax.ShapeDtypeStruct((B, N, 6), jnp.float32),
    )(rois, mrcnn_class, flat)
    data = jnp.transpose(data, (0, 2, 1))

    outs = pl.pallas_call(
        _nms_body,
        out_shape=[jax.ShapeDtypeStruct((B, K), jnp.float32)] * 6,
    )(data)
    osc, oy1, ox1, oy2, ox2, ocl = outs
    return jnp.stack([oy1, ox1, oy2, ox2, ocl, osc], axis=-1)


# speedup vs baseline: 18.5198x; 1.0819x over previous
import functools

import jax
import jax.numpy as jnp
from jax import lax
from jax.experimental import pallas as pl
from jax.experimental.pallas import tpu as pltpu
from jax.experimental.pallas import tpu_sc as plsc

B = 8
N = 1000
C = 81
K = 100
MIN_CONF = 0.7
NMS_THR = 0.3
N1 = 1024
NCH = N1 // 16
NC = 96
BIG = 2**30


def _dense_body(rois_ref, probs_ref, flat_ref, out_ref):
    probs = probs_ref[0]
    rois = rois_ref[0]
    flat = flat_ref[0]

    m = jnp.max(probs, axis=1, keepdims=True)
    iota_c = lax.broadcasted_iota(jnp.int32, (N, C), 1)
    cid = jnp.min(jnp.where(probs == m, iota_c, C), axis=1, keepdims=True)

    iota_f = lax.broadcasted_iota(jnp.int32, (N, 4 * C), 1)
    base = cid * 4
    d = []
    for k in range(4):
        mk = iota_f == (base + k)
        d.append(jnp.sum(jnp.where(mk, flat, 0.0), axis=1, keepdims=True))
    dy = d[0] * 0.1
    dx = d[1] * 0.1
    dh = d[2] * 0.2
    dw = d[3] * 0.2

    ry1 = rois[:, 0:1]
    rx1 = rois[:, 1:2]
    ry2 = rois[:, 2:3]
    rx2 = rois[:, 3:4]
    h = ry2 - ry1
    w = rx2 - rx1
    cy = ry1 + 0.5 * h
    cx = rx1 + 0.5 * w
    cy = cy + dy * h
    cx = cx + dx * w
    h = h * jnp.exp(dh)
    w = w * jnp.exp(dw)
    y1 = jnp.clip(cy - 0.5 * h, 0.0, 1.0)
    x1 = jnp.clip(cx - 0.5 * w, 0.0, 1.0)
    y2 = jnp.clip(cy + 0.5 * h, 0.0, 1.0)
    x2 = jnp.clip(cx + 0.5 * w, 0.0, 1.0)

    valid = (cid > 0) & (m >= MIN_CONF)
    sc = jnp.where(valid, m, -1.0)
    clsf = cid.astype(jnp.float32)

    cols = jnp.concatenate(
        [sc, y1, x1, y2, x2, clsf], axis=1)
    out_ref[0] = cols


def _sc_nms_body(data_hbm, out_hbm,
                 dsc, dy1, dx1, dy2, dx2, dcl,
                 bsc, by1, bx1, by2, bx2, bcl, bidx,
                 ny1, nx1, ny2, nx2, area,
                 counts, bases, wbase, cmax, ctie, obuf):
    wid = lax.axis_index("s") * 2 + lax.axis_index("c")

    @pl.when(wid < B)
    def _():
        pltpu.sync_copy(data_hbm.at[wid, 0], dsc)
        pltpu.sync_copy(data_hbm.at[wid, 1], dy1)
        pltpu.sync_copy(data_hbm.at[wid, 2], dx1)
        pltpu.sync_copy(data_hbm.at[wid, 3], dy2)
        pltpu.sync_copy(data_hbm.at[wid, 4], dx2)
        pltpu.sync_copy(data_hbm.at[wid, 5], dcl)

        iota = lax.iota(jnp.int32, 16)
        zeros16i = jnp.zeros((16,), jnp.int32)
        zeros16f = jnp.zeros((16,), jnp.float32)

        def chunk_rank(q):
            cls = dcl[pl.ds(q * 16, 16)].astype(jnp.int32)
            rank = jnp.zeros((16,), jnp.int32)
            after = jnp.zeros((16,), jnp.int32)
            one = jnp.full((16,), 1, jnp.int32)
            zero = jnp.zeros((16,), jnp.int32)
            for j in range(1, 16):
                dn = cls[jnp.maximum(iota - j, 0)]
                up = cls[jnp.minimum(iota + j, 15)]
                rank = rank + jnp.where((dn == cls) & (iota >= j), one, zero)
                after = after + jnp.where((up == cls) & (iota < 16 - j),
                                          one, zero)
            is_last = after == 0
            return cls, rank, is_last

        for t in range(NC // 16):
            counts[pl.ds(t * 16, 16)] = zeros16i

        def body1(q, c):
            cls, rank, is_last = chunk_rank(q)
            plsc.addupdate_scatter(counts, [cls], rank + 1, mask=is_last)
            return c

        lax.fori_loop(0, NCH, body1, 0)

        run = jnp.int32(0)
        for t in range(NC // 16):
            v = counts[pl.ds(t * 16, 16)]
            cs = plsc.cumsum(v)
            ex = cs - v + run
            bases[pl.ds(t * 16, 16)] = ex
            wbase[pl.ds(t * 16, 16)] = ex
            run = run + jnp.max(cs)

        def body3(q, c):
            cls, rank, is_last = chunk_rank(q)
            pos = plsc.load_gather(wbase, [cls]) + rank
            b16 = q * 16
            plsc.store_scatter(bsc, [pos], dsc[pl.ds(b16, 16)])
            plsc.store_scatter(by1, [pos], dy1[pl.ds(b16, 16)])
            plsc.store_scatter(bx1, [pos], dx1[pl.ds(b16, 16)])
            plsc.store_scatter(by2, [pos], dy2[pl.ds(b16, 16)])
            plsc.store_scatter(bx2, [pos], dx2[pl.ds(b16, 16)])
            plsc.store_scatter(bcl, [pos], dcl[pl.ds(b16, 16)])
            plsc.store_scatter(bidx, [pos], b16 + iota)
            plsc.addupdate_scatter(wbase, [cls], rank + 1, mask=is_last)
            return c

        lax.fori_loop(0, NCH, body3, 0)

        def body4(q, c):
            b16 = q * 16
            cl = bcl[pl.ds(b16, 16)]
            off = cl * 10.0
            v1 = by1[pl.ds(b16, 16)] + off
            u1 = bx1[pl.ds(b16, 16)] + off
            v2 = by2[pl.ds(b16, 16)] + off
            u2 = bx2[pl.ds(b16, 16)] + off
            ny1[pl.ds(b16, 16)] = v1
            nx1[pl.ds(b16, 16)] = u1
            ny2[pl.ds(b16, 16)] = v2
            nx2[pl.ds(b16, 16)] = u2
            area[pl.ds(b16, 16)] = (v2 - v1) * (u2 - u1)
            s = bsc[pl.ds(b16, 16)]
            mc = jnp.max(s)
            bi = bidx[pl.ds(b16, 16)]
            tc = jnp.min(jnp.where(s == mc, bi, BIG))
            lane0 = iota == 0
            qv = jnp.full((16,), q, jnp.int32)
            plsc.store_scatter(cmax, [qv], jnp.full((16,), mc, jnp.float32),
                               mask=lane0)
            plsc.store_scatter(ctie, [qv], jnp.full((16,), tc, jnp.int32),
                               mask=lane0)
            return c

        lax.fori_loop(0, NCH, body4, 0)

        def bodyz(q, c):
            obuf[pl.ds(q * 16, 16)] = zeros16f
            return c

        lax.fori_loop(0, K, bodyz, 0)

        def global_max():
            c0 = cmax[pl.ds(0, 16)]
            c1 = cmax[pl.ds(16, 16)]
            c2 = cmax[pl.ds(32, 16)]
            c3 = cmax[pl.ds(48, 16)]
            return jnp.max(jnp.maximum(jnp.maximum(c0, c1),
                                       jnp.maximum(c2, c3)))

        def wcond(st):
            k, m = st
            return (k < K) & (m > 0.0)

        def wbody(st):
            k, m = st
            c0 = cmax[pl.ds(0, 16)]
            c1 = cmax[pl.ds(16, 16)]
            c2 = cmax[pl.ds(32, 16)]
            c3 = cmax[pl.ds(48, 16)]
            t0 = ctie[pl.ds(0, 16)]
            t1 = ctie[pl.ds(16, 16)]
            t2 = ctie[pl.ds(32, 16)]
            t3 = ctie[pl.ds(48, 16)]
            big = jnp.full((16,), BIG, jnp.int32)
            a0 = jnp.where(c0 == m, t0, big)
            a1 = jnp.where(c1 == m, t1, big)
            a2 = jnp.where(c2 == m, t2, big)
            a3 = jnp.where(c3 == m, t3, big)
            tmin = jnp.min(jnp.minimum(jnp.minimum(a0, a1),
                                       jnp.minimum(a2, a3)))
            q0 = jnp.where((c0 == m) & (t0 == tmin), iota, big)
            q1 = jnp.where((c1 == m) & (t1 == tmin), iota + 16, big)
            q2 = jnp.where((c2 == m) & (t2 == tmin), iota + 32, big)
            q3 = jnp.where((c3 == m) & (t3 == tmin), iota + 48, big)
            qs = jnp.min(jnp.minimum(jnp.minimum(q0, q1),
                                     jnp.minimum(q2, q3)))
            qs = jnp.clip(qs, 0, NCH - 1)
            b16 = qs * 16
            s = bsc[pl.ds(b16, 16)]
            bi = bidx[pl.ds(b16, 16)]
            ps = jnp.min(jnp.where((s == m) & (bi == tmin), iota, 15))
            g = b16 + ps
            gv = jnp.full((16,), g, jnp.int32)
            lane0 = iota == 0
            vy1 = plsc.load_gather(by1, [gv])
            vx1 = plsc.load_gather(bx1, [gv])
            vy2 = plsc.load_gather(by2, [gv])
            vx2 = plsc.load_gather(bx2, [gv])
            vcl = plsc.load_gather(bcl, [gv])
            soff = vcl * 10.0
            sny1 = vy1 + soff
            snx1 = vx1 + soff
            sny2 = vy2 + soff
            snx2 = vx2 + soff
            sarea = (sny2 - sny1) * (snx2 - snx1)
            out16 = jnp.where(iota == 0, vy1, 0.0)
            out16 = jnp.where(iota == 1, vx1, out16)
            out16 = jnp.where(iota == 2, vy2, out16)
            out16 = jnp.where(iota == 3, vx2, out16)
            out16 = jnp.where(iota == 4, vcl, out16)
            out16 = jnp.where(iota == 5, jnp.full((16,), m), out16)
            obuf[pl.ds(k * 16, 16)] = out16
            civ = jnp.clip(vcl.astype(jnp.int32), 0, NC - 1)
            a = jnp.max(plsc.load_gather(bases, [civ]))
            bnd = a + jnp.max(plsc.load_gather(counts, [civ]))
            plsc.store_scatter(bsc, [gv],
                               jnp.full((16,), -1.0, jnp.float32),
                               mask=lane0)
            qa = jnp.clip(lax.div(a, 16), 0, NCH - 1)
            qb = jnp.clip(lax.div(bnd + 15, 16), 0, NCH)

            def sbody(qq, c):
                o = qq * 16
                gi = o + iota
                sv = bsc[pl.ds(o, 16)]
                yy1 = jnp.maximum(ny1[pl.ds(o, 16)], sny1)
                xx1 = jnp.maximum(nx1[pl.ds(o, 16)], snx1)
                yy2 = jnp.minimum(ny2[pl.ds(o, 16)], sny2)
                xx2 = jnp.minimum(nx2[pl.ds(o, 16)], snx2)
                inter = (jnp.maximum(yy2 - yy1, 0.0)
                         * jnp.maximum(xx2 - xx1, 0.0))
                union = sarea + area[pl.ds(o, 16)] - inter
                iou = inter / jnp.maximum(union, 1e-8)
                kill = ((gi >= a) & (gi < bnd) & (gi != g)
                        & (iou > NMS_THR))
                s2 = jnp.where(kill, -1.0, sv)
                bsc[pl.ds(o, 16)] = s2
                mc = jnp.max(s2)
                tc = jnp.min(jnp.where(s2 == mc, bidx[pl.ds(o, 16)], BIG))
                qv = jnp.full((16,), qq, jnp.int32)
                plsc.store_scatter(cmax, [qv],
                                   jnp.full((16,), mc, jnp.float32),
                                   mask=lane0)
                plsc.store_scatter(ctie, [qv],
                                   jnp.full((16,), tc, jnp.int32),
                                   mask=lane0)
                return c

            lax.fori_loop(qa, qb, sbody, 0)
            return (k + 1, global_max())

        lax.while_loop(wcond, wbody, (jnp.int32(0), global_max()))
        pltpu.sync_copy(obuf, out_hbm.at[wid])


def _sc_nms(data):
    mesh = plsc.VectorSubcoreMesh(core_axis_name="c", subcore_axis_name="s")
    f32 = jnp.float32
    i32 = jnp.int32
    run = functools.partial(
        pl.kernel,
        mesh=mesh,
        compiler_params=pltpu.CompilerParams(needs_layout_passes=False),
        out_type=jax.ShapeDtypeStruct((B, K * 16), f32),
        scratch_types=[
            pltpu.VMEM((N1,), f32),
            pltpu.VMEM((N1,), f32),
            pltpu.VMEM((N1,), f32),
            pltpu.VMEM((N1,), f32),
            pltpu.VMEM((N1,), f32),
            pltpu.VMEM((N1,), f32),
            pltpu.VMEM((N1,), f32),
            pltpu.VMEM((N1,), f32),
            pltpu.VMEM((N1,), f32),
            pltpu.VMEM((N1,), f32),
            pltpu.VMEM((N1,), f32),
            pltpu.VMEM((N1,), f32),
            pltpu.VMEM((N1,), i32),
            pltpu.VMEM((N1,), f32),
            pltpu.VMEM((N1,), f32),
            pltpu.VMEM((N1,), f32),
            pltpu.VMEM((N1,), f32),
            pltpu.VMEM((N1,), f32),
            pltpu.VMEM((NC,), i32),
            pltpu.VMEM((NC,), i32),
            pltpu.VMEM((NC,), i32),
            pltpu.VMEM((NCH,), f32),
            pltpu.VMEM((NCH,), i32),
            pltpu.VMEM((K * 16,), f32),
        ],
    )(_sc_nms_body)
    return run(data)


def kernel(rois, mrcnn_class, mrcnn_bbox):
    flat = mrcnn_bbox.reshape(B, N, 4 * C)
    data = pl.pallas_call(
        _dense_body,
        grid=(B,),
        in_specs=[
            pl.BlockSpec((1, N, 4), lambda b: (b, 0, 0)),
            pl.BlockSpec((1, N, C), lambda b: (b, 0, 0)),
            pl.BlockSpec((1, N, 4 * C), lambda b: (b, 0, 0)),
        ],
        out_specs=pl.BlockSpec((1, N, 6), lambda b: (b, 0, 0)),
        out_shape=jax.ShapeDtypeStruct((B, N, 6), jnp.float32),
    )(rois, mrcnn_class, flat)
    data = jnp.transpose(data, (0, 2, 1))
    pad = jnp.concatenate(
        [jnp.full((B, 1, N1 - N), -1.0, jnp.float32),
         jnp.zeros((B, 5, N1 - N), jnp.float32)], axis=1)
    data = jnp.concatenate([data, pad], axis=2)
    out = _sc_nms(data)
    det = out.reshape(B, K, 16)[:, :, :6]
    return det
